# baseline (speedup 1.0000x reference)
import functools

import jax
import jax.numpy as jnp
from jax import lax
from jax.experimental import pallas as pl
from jax.experimental.pallas import tpu as pltpu
from jax.experimental.pallas import tpu_sc as plsc

N_TOK = 16384
DM = 2048
NE = 16
TN = 2048

NCHUNK = 2
HALF = N_TOK // NCHUNK
NW = 32
TPWC = HALF // NW
GROUPSC = TPWC // 16
L = 16


def _logits_body(x_ref, w_ref, b_ref, lg_ref):
    lg = jax.lax.dot_general(
        x_ref[...], w_ref[...], (((1,), (1,)), ((), ())),
        preferred_element_type=jnp.float32,
    ) + b_ref[...]
    lg_ref[...] = lg.T


def _logits_tc(x, W, b, off):
    grid = HALF // TN
    return pl.pallas_call(
        _logits_body,
        grid=(grid,),
        in_specs=[
            pl.BlockSpec((TN, DM), lambda i, off=off: (i + off, 0)),
            pl.BlockSpec((NE, DM), lambda i: (0, 0)),
            pl.BlockSpec((1, NE), lambda i: (0, 0)),
        ],
        out_specs=pl.BlockSpec((NE, TN), lambda i: (0, i)),
        out_shape=jax.ShapeDtypeStruct((NE, HALF), jnp.float32),
        compiler_params=pltpu.CompilerParams(
            dimension_semantics=("arbitrary",)
        ),
    )(x, W, b.reshape(1, NE))


_mesh = plsc.VectorSubcoreMesh(core_axis_name="c", subcore_axis_name="s")


@functools.partial(
    pl.kernel,
    out_type=[
        jax.ShapeDtypeStruct((NE, HALF), jnp.float32),
        jax.ShapeDtypeStruct((2, HALF), jnp.int32),
    ],
    mesh=_mesh,
    scratch_types=[
        pltpu.VMEM((NE, TPWC), jnp.float32),
        pltpu.VMEM((NE, TPWC), jnp.float32),
        pltpu.VMEM((TPWC,), jnp.int32),
        pltpu.VMEM((TPWC,), jnp.int32),
    ],
    compiler_params=pltpu.CompilerParams(needs_layout_passes=False),
)
def _route_sc(lg_hbm, cw_hbm, idx_hbm, lg_v, cw_v, i1_v, i2_v):
    wid = lax.axis_index("s") * 2 + lax.axis_index("c")
    base = wid * TPWC
    pltpu.sync_copy(lg_hbm.at[:, pl.ds(base, TPWC)], lg_v)

    neg = jnp.full((L,), -3.4e38, jnp.float32)
    zero_i = jnp.zeros((L,), jnp.int32)
    zero_f = jnp.zeros((L,), jnp.float32)

    def group(g, carry):
        t0 = g * L
        m1, m2 = neg, neg
        i1, i2 = zero_i, zero_i
        for e in range(NE):
            esp = jnp.full((L,), e, jnp.int32)
            le = lg_v[e, pl.ds(t0, L)]
            gt1 = le > m1
            gt2 = le > m2
            i2 = jnp.where(gt1, i1, jnp.where(gt2, esp, i2))
            m2 = jnp.where(gt1, m1, jnp.where(gt2, le, m2))
            i1 = jnp.where(gt1, esp, i1)
            m1 = jnp.where(gt1, le, m1)
        w1 = 1.0 / (1.0 + jnp.exp(m2 - m1))
        w2 = 1.0 - w1
        for e in range(NE):
            esp = jnp.full((L,), e, jnp.int32)
            val = jnp.where(i1 == esp, w1, jnp.where(i2 == esp, w2, zero_f))
            cw_v[e, pl.ds(t0, L)] = val
        i1_v[pl.ds(t0, L)] = i1
        i2_v[pl.ds(t0, L)] = i2
        return carry

    lax.fori_loop(0, GROUPSC, group, 0)
    pltpu.sync_copy(cw_v, cw_hbm.at[:, pl.ds(base, TPWC)])
    pltpu.sync_copy(i1_v, idx_hbm.at[0, pl.ds(base, TPWC)])
    pltpu.sync_copy(i2_v, idx_hbm.at[1, pl.ds(base, TPWC)])


def kernel(x, W, b):
    lg_a = _logits_tc(x, W, b, 0)
    cw_a, idx_a = _route_sc(lg_a)
    lg_b = _logits_tc(x, W, b, HALF // TN)
    cw_b, idx_b = _route_sc(lg_b)
    cw_t = jnp.concatenate([cw_a, cw_b], axis=1)
    idx_t = jnp.concatenate([idx_a, idx_b], axis=1)
    return (cw_t.T[..., None], idx_t.T, jnp.float32(0.0))


# speedup vs baseline: 1.4155x; 1.4155x over previous
import jax
import jax.numpy as jnp
from jax.experimental import pallas as pl
from jax.experimental.pallas import tpu as pltpu

N_TOK = 16384
DM = 2048
NE = 16
TN = 2048


def _gate_body(x_ref, w_ref, b_ref, cw_ref, idx_ref):
    x = x_ref[...]
    w = w_ref[...]
    logits = jax.lax.dot_general(
        x, w, (((1,), (1,)), ((), ())), preferred_element_type=jnp.float32
    ) + b_ref[...]
    eidx = jax.lax.broadcasted_iota(jnp.int32, logits.shape, 1)
    neg = jnp.float32(-3.4e38)
    m1 = jnp.max(logits, axis=1, keepdims=True)
    i1 = jnp.min(jnp.where(logits == m1, eidx, NE), axis=1, keepdims=True)
    l2 = jnp.where(eidx == i1, neg, logits)
    m2 = jnp.max(l2, axis=1, keepdims=True)
    i2 = jnp.min(jnp.where(l2 == m2, eidx, NE), axis=1, keepdims=True)
    w1 = 1.0 / (1.0 + jnp.exp(m2 - m1))
    w2 = 1.0 - w1
    cw = jnp.where(eidx == i1, w1, 0.0) + jnp.where(eidx == i2, w2, 0.0)
    cw_ref[...] = cw.T
    idx_ref[...] = jnp.concatenate([i1, i2], axis=1).T


def kernel(x, W, b):
    grid = N_TOK // TN
    cw_t, idx_t = pl.pallas_call(
        _gate_body,
        grid=(grid,),
        in_specs=[
            pl.BlockSpec((TN, DM), lambda i: (i, 0)),
            pl.BlockSpec((NE, DM), lambda i: (0, 0)),
            pl.BlockSpec((1, NE), lambda i: (0, 0)),
        ],
        out_specs=[
            pl.BlockSpec((NE, TN), lambda i: (0, i)),
            pl.BlockSpec((2, TN), lambda i: (0, i)),
        ],
        out_shape=[
            jax.ShapeDtypeStruct((NE, N_TOK), jnp.float32),
            jax.ShapeDtypeStruct((2, N_TOK), jnp.int32),
        ],
        compiler_params=pltpu.CompilerParams(
            dimension_semantics=("arbitrary",)
        ),
    )(x, W, b.reshape(1, NE))
    return (cw_t.T[..., None], idx_t.T, jnp.float32(0.0))
